# baseline (speedup 1.0000x reference)
import functools

import jax
import jax.numpy as jnp
from jax import lax
from jax.experimental import pallas as pl
from jax.experimental.pallas import tpu as pltpu
from jax.experimental.pallas import tpu_sc as plsc

NC = 2
NS = 16
NW = NC * NS
CHUNK = 128
NBUF = 4


def _build_call(B, S, V, H):
    rows_per_w = B * S // NW
    n_chunks = rows_per_w // CHUNK
    n_groups = n_chunks // NBUF
    mesh = plsc.VectorSubcoreMesh(core_axis_name="c", subcore_axis_name="s")

    def body(idx_hbm, tok_hbm, pos_hbm, out_hbm, idx_v, pos_v,
             d0, d1, d2, d3, g0, g1, g2, g3, s0, s1, s2, s3,
             p0, p1, p2, p3):
        sid = lax.axis_index("s")
        wid = sid * NC + lax.axis_index("c")
        base = wid * rows_per_w

        pltpu.sync_copy(idx_hbm.at[wid], idx_v)

        @pl.when(sid == 0)
        def _():
            pltpu.sync_copy(pos_hbm, pos_v)
        plsc.subcore_barrier()

        bufs = (d0, d1, d2, d3)
        gs = (g0, g1, g2, g3)
        ss = (s0, s1, s2, s3)
        ps = (p0, p1, p2, p3)

        def p_cp(j, b):
            off = pl.multiple_of(lax.rem(j * CHUNK, S), 8)
            return pltpu.make_async_copy(
                pos_v.at[pl.ds(off, CHUNK)], bufs[b], ps[b])

        def g_cp(j, b):
            return pltpu.make_async_copy(
                tok_hbm.at[idx_v.at[j]], bufs[b], gs[b])

        def s_cp(j, b):
            off = pl.multiple_of(base + j * CHUNK, 8)
            return pltpu.make_async_copy(
                bufs[b], out_hbm.at[pl.ds(off, CHUNK)], ss[b])

        def step(j, b, skip_s_wait=False, fire_p=True, fire_g=True):
            g_cp(j, b).wait()
            s_cp(j, b).start()
            if not skip_s_wait:
                s_cp(j - 2, (b - 2) % NBUF).wait()
            if fire_p:
                p_cp(j + 2, (b + 2) % NBUF).start()
            if fire_g:
                p_cp(j + 1, (b + 1) % NBUF).wait()
                g_cp(j + 1, (b + 1) % NBUF).start(add=True)

        p_cp(0, 0).start()
        p_cp(1, 1).start()
        p_cp(0, 0).wait()
        g_cp(0, 0).start(add=True)

        for b in range(NBUF):
            step(b, b, skip_s_wait=b < 2)

        def group(k, _):
            for b in range(NBUF):
                step(NBUF * k + b, b)
            return 0

        lax.fori_loop(1, n_groups - 1, group, 0)

        for b in range(NBUF):
            j = NBUF * (n_groups - 1) + b
            step(j, b, fire_p=j + 2 < n_chunks, fire_g=j + 1 < n_chunks)

        s_cp(n_chunks - 2, (NBUF - 2) % NBUF).wait()
        s_cp(n_chunks - 1, (NBUF - 1) % NBUF).wait()

    return pl.kernel(
        body,
        out_type=jax.ShapeDtypeStruct((B * S, H), jnp.float32),
        mesh=mesh,
        scratch_types=[
            pltpu.VMEM((n_chunks, CHUNK), jnp.int32),
            pltpu.VMEM_SHARED((S + CHUNK, H), jnp.float32),
            pltpu.VMEM((CHUNK, H), jnp.float32),
            pltpu.VMEM((CHUNK, H), jnp.float32),
            pltpu.VMEM((CHUNK, H), jnp.float32),
            pltpu.VMEM((CHUNK, H), jnp.float32),
        ] + [pltpu.SemaphoreType.DMA] * 12,
    )


@jax.jit
def kernel(input_ids, token_table, pos_table):
    B, S = input_ids.shape
    V, H = token_table.shape
    idx_r = input_ids.reshape(NW, -1, CHUNK).astype(jnp.int32)
    pos_ext = jnp.concatenate([pos_table[:S], pos_table[:CHUNK]], axis=0)
    out = _build_call(B, S, V, H)(idx_r, token_table, pos_ext)
    return out.reshape(B, S, H)


# speedup vs baseline: 1.2262x; 1.2262x over previous
import functools

import jax
import jax.numpy as jnp
from jax import lax
from jax.experimental import pallas as pl
from jax.experimental.pallas import tpu as pltpu
from jax.experimental.pallas import tpu_sc as plsc

NC = 2
NS = 16
NW = NC * NS
CHUNK = 128
NBUF = 5


def _build_call(B, S, V, H):
    rows_per_w = B * S // NW
    n_chunks = rows_per_w // CHUNK
    n_groups = n_chunks // NBUF
    mesh = plsc.VectorSubcoreMesh(core_axis_name="c", subcore_axis_name="s")

    def body(idx_hbm, tok_hbm, pos_hbm, out_hbm, idx_v, pos_v,
             d0, d1, d2, d3, d4, g0, g1, g2, g3, g4,
             s0, s1, s2, s3, s4, p0, p1, p2, p3, p4):
        sid = lax.axis_index("s")
        wid = sid * NC + lax.axis_index("c")
        base = wid * rows_per_w

        pltpu.sync_copy(idx_hbm.at[wid], idx_v)

        @pl.when(sid == 0)
        def _():
            pltpu.sync_copy(pos_hbm, pos_v)
        plsc.subcore_barrier()

        bufs = (d0, d1, d2, d3, d4)
        gs = (g0, g1, g2, g3, g4)
        ss = (s0, s1, s2, s3, s4)
        ps = (p0, p1, p2, p3, p4)

        def p_cp(j, b):
            off = pl.multiple_of(lax.rem(j * CHUNK, S), 8)
            return pltpu.make_async_copy(
                pos_v.at[pl.ds(off, CHUNK)], bufs[b], ps[b])

        def g_cp(j, b):
            return pltpu.make_async_copy(
                tok_hbm.at[idx_v.at[j]], bufs[b], gs[b])

        def s_cp(j, b):
            off = pl.multiple_of(base + j * CHUNK, 8)
            return pltpu.make_async_copy(
                bufs[b], out_hbm.at[pl.ds(off, CHUNK)], ss[b])

        def step(j, b, skip_s_wait=False, fire_p=True, fire_g=True):
            g_cp(j, b).wait()
            s_cp(j, b).start()
            if not skip_s_wait:
                s_cp(j - 2, (b - 2) % NBUF).wait()
            if fire_p:
                p_cp(j + 3, (b + 3) % NBUF).start()
            if fire_g:
                p_cp(j + 2, (b + 2) % NBUF).wait()
                g_cp(j + 2, (b + 2) % NBUF).start(add=True)

        p_cp(0, 0).start()
        p_cp(1, 1).start()
        p_cp(2, 2).start()
        p_cp(0, 0).wait()
        g_cp(0, 0).start(add=True)
        p_cp(1, 1).wait()
        g_cp(1, 1).start(add=True)

        for b in range(NBUF):
            step(b, b, skip_s_wait=b < 2)

        def group(k, _):
            for b in range(NBUF):
                step(NBUF * k + b, b)
            return 0

        lax.fori_loop(1, n_groups - 1, group, 0)

        for b in range(NBUF):
            j = NBUF * (n_groups - 1) + b
            step(j, b, fire_p=j + 3 < n_chunks, fire_g=j + 2 < n_chunks)

        s_cp(n_chunks - 2, (NBUF - 2) % NBUF).wait()
        s_cp(n_chunks - 1, (NBUF - 1) % NBUF).wait()

    return pl.kernel(
        body,
        out_type=jax.ShapeDtypeStruct((B * S, H), jnp.float32),
        mesh=mesh,
        scratch_types=[
            pltpu.VMEM((n_chunks, CHUNK), jnp.int32),
            pltpu.VMEM_SHARED((S + CHUNK, H), jnp.float32),
            pltpu.VMEM((CHUNK, H), jnp.float32),
            pltpu.VMEM((CHUNK, H), jnp.float32),
            pltpu.VMEM((CHUNK, H), jnp.float32),
            pltpu.VMEM((CHUNK, H), jnp.float32),
            pltpu.VMEM((CHUNK, H), jnp.float32),
        ] + [pltpu.SemaphoreType.DMA] * 15,
    )


@jax.jit
def kernel(input_ids, token_table, pos_table):
    B, S = input_ids.shape
    V, H = token_table.shape
    idx_r = input_ids.reshape(NW, -1, CHUNK).astype(jnp.int32)
    pos_ext = jnp.concatenate([pos_table[:S], pos_table[:CHUNK]], axis=0)
    out = _build_call(B, S, V, H)(idx_r, token_table, pos_ext)
    return out.reshape(B, S, H)


# speedup vs baseline: 1.2272x; 1.0008x over previous
import functools

import jax
import jax.numpy as jnp
from jax import lax
from jax.experimental import pallas as pl
from jax.experimental.pallas import tpu as pltpu
from jax.experimental.pallas import tpu_sc as plsc

NC = 2
NS = 16
NW = NC * NS
CHUNK = 128
NBUF = 5


def _build_call(B, S, V, H):
    rows_per_w = B * S // NW
    n_chunks = rows_per_w // CHUNK
    n_groups = n_chunks // NBUF
    mesh = plsc.VectorSubcoreMesh(core_axis_name="c", subcore_axis_name="s")

    def body(idx_hbm, tok_hbm, pos_hbm, out_hbm, idx_v, pos_v,
             d0, d1, d2, d3, d4, g0, g1, g2, g3, g4,
             s0, s1, s2, s3, s4, p0, p1, p2, p3, p4):
        sid = lax.axis_index("s")
        wid = sid * NC + lax.axis_index("c")
        base = wid * rows_per_w

        pltpu.sync_copy(idx_hbm.at[wid], idx_v)

        @pl.when(sid == 0)
        def _():
            pltpu.sync_copy(pos_hbm, pos_v)
        plsc.subcore_barrier()

        bufs = (d0, d1, d2, d3, d4)
        gs = (g0, g1, g2, g3, g4)
        ss = (s0, s1, s2, s3, s4)
        ps = (p0, p1, p2, p3, p4)

        def p_cp(j, b):
            off = pl.multiple_of(lax.rem(j * CHUNK, S), 8)
            return pltpu.make_async_copy(
                pos_v.at[pl.ds(off, CHUNK)], bufs[b], ps[b])

        def g_cps(j, b):
            return [pltpu.make_async_copy(
                tok_hbm.at[idx_v.at[j, pl.ds(h * (CHUNK // 2), CHUNK // 2)]],
                bufs[b].at[pl.ds(h * (CHUNK // 2), CHUNK // 2)],
                gs[b]) for h in range(2)]

        def s_cp(j, b):
            off = pl.multiple_of(base + j * CHUNK, 8)
            return pltpu.make_async_copy(
                bufs[b], out_hbm.at[pl.ds(off, CHUNK)], ss[b])

        def step(j, b, skip_s_wait=False, fire_p=True, fire_g=True):
            for cp in g_cps(j, b):
                cp.wait()
            s_cp(j, b).start()
            if not skip_s_wait:
                s_cp(j - 2, (b - 2) % NBUF).wait()
            if fire_p:
                p_cp(j + 3, (b + 3) % NBUF).start()
            if fire_g:
                p_cp(j + 2, (b + 2) % NBUF).wait()
                for cp in g_cps(j + 2, (b + 2) % NBUF):
                    cp.start(add=True)

        p_cp(0, 0).start()
        p_cp(1, 1).start()
        p_cp(2, 2).start()
        p_cp(0, 0).wait()
        for cp in g_cps(0, 0):
            cp.start(add=True)
        p_cp(1, 1).wait()
        for cp in g_cps(1, 1):
            cp.start(add=True)

        for b in range(NBUF):
            step(b, b, skip_s_wait=b < 2)

        def group(k, _):
            for b in range(NBUF):
                step(NBUF * k + b, b)
            return 0

        lax.fori_loop(1, n_groups - 1, group, 0)

        for b in range(NBUF):
            j = NBUF * (n_groups - 1) + b
            step(j, b, fire_p=j + 3 < n_chunks, fire_g=j + 2 < n_chunks)

        s_cp(n_chunks - 2, (NBUF - 2) % NBUF).wait()
        s_cp(n_chunks - 1, (NBUF - 1) % NBUF).wait()

    return pl.kernel(
        body,
        out_type=jax.ShapeDtypeStruct((B * S, H), jnp.float32),
        mesh=mesh,
        scratch_types=[
            pltpu.VMEM((n_chunks, CHUNK), jnp.int32),
            pltpu.VMEM_SHARED((S + CHUNK, H), jnp.float32),
            pltpu.VMEM((CHUNK, H), jnp.float32),
            pltpu.VMEM((CHUNK, H), jnp.float32),
            pltpu.VMEM((CHUNK, H), jnp.float32),
            pltpu.VMEM((CHUNK, H), jnp.float32),
            pltpu.VMEM((CHUNK, H), jnp.float32),
        ] + [pltpu.SemaphoreType.DMA] * 15,
    )


@jax.jit
def kernel(input_ids, token_table, pos_table):
    B, S = input_ids.shape
    V, H = token_table.shape
    idx_r = input_ids.reshape(NW, -1, CHUNK).astype(jnp.int32)
    pos_ext = jnp.concatenate([pos_table[:S], pos_table[:CHUNK]], axis=0)
    out = _build_call(B, S, V, H)(idx_r, token_table, pos_ext)
    return out.reshape(B, S, H)


# speedup vs baseline: 1.2679x; 1.0332x over previous
import functools

import jax
import jax.numpy as jnp
from jax import lax
from jax.experimental import pallas as pl
from jax.experimental.pallas import tpu as pltpu
from jax.experimental.pallas import tpu_sc as plsc

NC = 2
NS = 16
NW = NC * NS
CHUNK = 128
NBUF = 5


def _build_call(B, S, V, H):
    rows_per_w = B * S // NW
    n_chunks = rows_per_w // CHUNK
    n_groups = n_chunks // NBUF
    mesh = plsc.VectorSubcoreMesh(core_axis_name="c", subcore_axis_name="s")

    def body(idx_hbm, tok_hbm, pos_hbm, out_hbm, idx_v, pos_v,
             d0, d1, d2, d3, d4, g0, g1, g2, g3, g4,
             s0, s1, s2, s3, s4, p0, p1, p2, p3, p4):
        sid = lax.axis_index("s")
        wid = sid * NC + lax.axis_index("c")
        base = wid * rows_per_w

        pltpu.sync_copy(idx_hbm.at[wid], idx_v)

        @pl.when(sid == 0)
        def _():
            pltpu.sync_copy(pos_hbm, pos_v)
        plsc.subcore_barrier()

        bufs = (d0, d1, d2, d3, d4)
        gs = (g0, g1, g2, g3, g4)
        ss = (s0, s1, s2, s3, s4)
        ps = (p0, p1, p2, p3, p4)

        def p_cp(j, b):
            j = lax.rem(j + sid * 13, n_chunks)
            off = pl.multiple_of(lax.rem(j * CHUNK, S), 8)
            return pltpu.make_async_copy(
                pos_v.at[pl.ds(off, CHUNK)], bufs[b], ps[b])

        def g_cps(j, b):
            j = lax.rem(j + sid * 13, n_chunks)
            return [pltpu.make_async_copy(
                tok_hbm.at[idx_v.at[j, pl.ds(h * (CHUNK // 2), CHUNK // 2)]],
                bufs[b].at[pl.ds(h * (CHUNK // 2), CHUNK // 2)],
                gs[b]) for h in range(2)]

        def s_cp(j, b):
            j = lax.rem(j + sid * 13, n_chunks)
            off = pl.multiple_of(base + j * CHUNK, 8)
            return pltpu.make_async_copy(
                bufs[b], out_hbm.at[pl.ds(off, CHUNK)], ss[b])

        def step(j, b, skip_s_wait=False, fire_p=True, fire_g=True):
            for cp in g_cps(j, b):
                cp.wait()
            s_cp(j, b).start()
            if not skip_s_wait:
                s_cp(j - 2, (b - 2) % NBUF).wait()
            if fire_p:
                p_cp(j + 3, (b + 3) % NBUF).start()
            if fire_g:
                p_cp(j + 2, (b + 2) % NBUF).wait()
                for cp in g_cps(j + 2, (b + 2) % NBUF):
                    cp.start(add=True)

        p_cp(0, 0).start()
        p_cp(1, 1).start()
        p_cp(2, 2).start()
        p_cp(0, 0).wait()
        for cp in g_cps(0, 0):
            cp.start(add=True)
        p_cp(1, 1).wait()
        for cp in g_cps(1, 1):
            cp.start(add=True)

        for b in range(NBUF):
            step(b, b, skip_s_wait=b < 2)

        def group(k, _):
            for b in range(NBUF):
                step(NBUF * k + b, b)
            return 0

        lax.fori_loop(1, n_groups - 1, group, 0)

        for b in range(NBUF):
            j = NBUF * (n_groups - 1) + b
            step(j, b, fire_p=j + 3 < n_chunks, fire_g=j + 2 < n_chunks)

        s_cp(n_chunks - 2, (NBUF - 2) % NBUF).wait()
        s_cp(n_chunks - 1, (NBUF - 1) % NBUF).wait()

    return pl.kernel(
        body,
        out_type=jax.ShapeDtypeStruct((B * S, H), jnp.float32),
        mesh=mesh,
        scratch_types=[
            pltpu.VMEM((n_chunks, CHUNK), jnp.int32),
            pltpu.VMEM_SHARED((S + CHUNK, H), jnp.float32),
            pltpu.VMEM((CHUNK, H), jnp.float32),
            pltpu.VMEM((CHUNK, H), jnp.float32),
            pltpu.VMEM((CHUNK, H), jnp.float32),
            pltpu.VMEM((CHUNK, H), jnp.float32),
            pltpu.VMEM((CHUNK, H), jnp.float32),
        ] + [pltpu.SemaphoreType.DMA] * 15,
    )


@jax.jit
def kernel(input_ids, token_table, pos_table):
    B, S = input_ids.shape
    V, H = token_table.shape
    idx_r = input_ids.reshape(NW, -1, CHUNK).astype(jnp.int32)
    pos_ext = jnp.concatenate([pos_table[:S], pos_table[:CHUNK]], axis=0)
    out = _build_call(B, S, V, H)(idx_r, token_table, pos_ext)
    return out.reshape(B, S, H)


# speedup vs baseline: 1.2706x; 1.0021x over previous
import functools

import jax
import jax.numpy as jnp
from jax import lax
from jax.experimental import pallas as pl
from jax.experimental.pallas import tpu as pltpu
from jax.experimental.pallas import tpu_sc as plsc

NC = 2
NS = 16
NW = NC * NS
CHUNK = 128
NBUF = 5


def _build_call(B, S, V, H):
    rows_per_w = B * S // NW
    n_chunks = rows_per_w // CHUNK
    n_groups = n_chunks // NBUF
    mesh = plsc.VectorSubcoreMesh(core_axis_name="c", subcore_axis_name="s")

    def body(idx_hbm, tok_hbm, pos_hbm, out_hbm, idx_v, pos_v,
             d0, d1, d2, d3, d4, g0, g1, g2, g3, g4,
             s0, s1, s2, s3, s4, p0, p1, p2, p3, p4):
        sid = lax.axis_index("s")
        wid = sid * NC + lax.axis_index("c")
        base = wid * rows_per_w

        pltpu.sync_copy(idx_hbm.at[wid], idx_v)

        @pl.when(sid == 0)
        def _():
            pltpu.sync_copy(pos_hbm, pos_v)
        plsc.subcore_barrier()

        bufs = (d0, d1, d2, d3, d4)
        gs = (g0, g1, g2, g3, g4)
        ss = (s0, s1, s2, s3, s4)
        ps = (p0, p1, p2, p3, p4)

        def p_cp(j, b):
            j = lax.rem(j + sid * 13, n_chunks)
            off = pl.multiple_of(lax.rem(j * CHUNK, S), 8)
            return pltpu.make_async_copy(
                pos_v.at[pl.ds(off, CHUNK)], bufs[b], ps[b])

        def g_cps(j, b):
            j = lax.rem(j + sid * 13, n_chunks)
            return [pltpu.make_async_copy(
                tok_hbm.at[idx_v.at[j, pl.ds(h * (CHUNK // 2), CHUNK // 2)]],
                bufs[b].at[pl.ds(h * (CHUNK // 2), CHUNK // 2)],
                gs[b]) for h in range(2)]

        def s_cp(j, b):
            j = lax.rem(j + sid * 13, n_chunks)
            off = pl.multiple_of(base + j * CHUNK, 8)
            return pltpu.make_async_copy(
                bufs[b], out_hbm.at[pl.ds(off, CHUNK)], ss[b])

        def step(j, b, skip_s_wait=False, fire_p=True, fire_g=True):
            for cp in g_cps(j, b):
                cp.wait()
            if fire_g:
                p_cp(j + 2, (b + 2) % NBUF).wait()
                for cp in g_cps(j + 2, (b + 2) % NBUF):
                    cp.start(add=True)
            s_cp(j, b).start()
            if not skip_s_wait:
                s_cp(j - 2, (b - 2) % NBUF).wait()
            if fire_p:
                p_cp(j + 3, (b + 3) % NBUF).start()

        p_cp(0, 0).start()
        p_cp(1, 1).start()
        p_cp(2, 2).start()
        p_cp(0, 0).wait()
        for cp in g_cps(0, 0):
            cp.start(add=True)
        p_cp(1, 1).wait()
        for cp in g_cps(1, 1):
            cp.start(add=True)

        for b in range(NBUF):
            step(b, b, skip_s_wait=b < 2)

        def group(k, _):
            for b in range(NBUF):
                step(NBUF * k + b, b)
            return 0

        lax.fori_loop(1, n_groups - 1, group, 0)

        for b in range(NBUF):
            j = NBUF * (n_groups - 1) + b
            step(j, b, fire_p=j + 3 < n_chunks, fire_g=j + 2 < n_chunks)

        s_cp(n_chunks - 2, (NBUF - 2) % NBUF).wait()
        s_cp(n_chunks - 1, (NBUF - 1) % NBUF).wait()

    return pl.kernel(
        body,
        out_type=jax.ShapeDtypeStruct((B * S, H), jnp.float32),
        mesh=mesh,
        scratch_types=[
            pltpu.VMEM((n_chunks, CHUNK), jnp.int32),
            pltpu.VMEM_SHARED((S + CHUNK, H), jnp.float32),
            pltpu.VMEM((CHUNK, H), jnp.float32),
            pltpu.VMEM((CHUNK, H), jnp.float32),
            pltpu.VMEM((CHUNK, H), jnp.float32),
            pltpu.VMEM((CHUNK, H), jnp.float32),
            pltpu.VMEM((CHUNK, H), jnp.float32),
        ] + [pltpu.SemaphoreType.DMA] * 15,
    )


@jax.jit
def kernel(input_ids, token_table, pos_table):
    B, S = input_ids.shape
    V, H = token_table.shape
    idx_r = input_ids.reshape(NW, -1, CHUNK).astype(jnp.int32)
    pos_ext = jnp.concatenate([pos_table[:S], pos_table[:CHUNK]], axis=0)
    out = _build_call(B, S, V, H)(idx_r, token_table, pos_ext)
    return out.reshape(B, S, H)
